# Initial kernel scaffold; baseline (speedup 1.0000x reference)
import jax
import jax.numpy as jnp
from jax.experimental import pallas as pl


def kernel(x, edge_index, batch, gene_features, params):
    raise NotImplementedError("write your pallas kernel here")


# speedup vs baseline: 2.5062x; 2.5062x over previous
import jax
import jax.numpy as jnp
from jax.experimental import pallas as pl
from jax.experimental.pallas import tpu as pltpu

N = 50000
B = 1024
H = 128
L = 3
EPS = 1e-5


def _node_embed_body(x_ref, w_ref, b_ref, o_ref):
    o_ref[...] = jax.nn.relu(
        jnp.dot(x_ref[...], w_ref[...], preferred_element_type=jnp.float32)
        + b_ref[...]
    )


def _node_embed(x, W, b):
    return pl.pallas_call(
        _node_embed_body,
        out_shape=jax.ShapeDtypeStruct((N, H), jnp.float32),
    )(x, W, b[None, :])


def _bn(x, g, b):
    mu = jnp.mean(x, axis=0)
    var = jnp.var(x, axis=0)
    return (x - mu) / jnp.sqrt(var + EPS) * g + b


def kernel(x, edge_index, batch, gene_features, params):
    n = x.shape[0]
    src, dst = edge_index[0], edge_index[1]
    h = _node_embed(x, params['node_W'], params['node_b'])
    loop = jnp.arange(n, dtype=src.dtype)
    deg = jnp.zeros((n,), jnp.float32).at[dst].add(1.0) + 1.0
    dinv = 1.0 / jnp.sqrt(deg)
    for i in range(L):
        hp = (h @ params['gcn_W'][i]) * dinv[:, None]
        agg = hp + jnp.zeros((n, H), jnp.float32).at[dst].add(hp[src])
        hn = agg * dinv[:, None] + params['gcn_b'][i]
        hn = _bn(hn, params['bn_g'][i], params['bn_b'][i])
        hn = jax.nn.relu(hn)
        h = h + hn if i > 0 else hn
    ones = jnp.ones((n,), jnp.float32)
    counts = jax.ops.segment_sum(ones, batch, num_segments=B)
    sums = jax.ops.segment_sum(h, batch, num_segments=B)
    mean_pool = sums / jnp.clip(counts, 1.0)[:, None]
    max_pool = jax.ops.segment_max(h, batch, num_segments=B)
    max_pool = jnp.where(jnp.isfinite(max_pool), max_pool, 0.0)
    graph_repr = jnp.concatenate([mean_pool, max_pool], axis=1)
    d1 = jax.nn.relu(graph_repr @ params['proj_W1'] + params['proj_b1'])
    drug_embedding = d1 @ params['proj_W2'] + params['proj_b2']
    g1 = gene_features @ params['gene_W1'] + params['gene_b1']
    g1 = jax.nn.relu(_bn(g1, params['gene_bn_g'], params['gene_bn_b']))
    gene_embedding = jax.nn.relu(g1 @ params['gene_W2'] + params['gene_b2'])
    combined = jnp.concatenate([drug_embedding, gene_embedding], axis=1)
    p1 = combined @ params['head_W1'] + params['head_b1']
    p1 = jax.nn.relu(_bn(p1, params['head_bn_g'], params['head_bn_b']))
    p2 = jax.nn.relu(p1 @ params['head_W2'] + params['head_b2'])
    predictions = p2 @ params['head_W3'] + params['head_b3']
    return predictions


# speedup vs baseline: 8.9723x; 3.5800x over previous
import jax
import jax.numpy as jnp
from jax import lax
from jax.experimental import pallas as pl
from jax.experimental.pallas import tpu as pltpu
from jax.experimental.pallas import tpu_sc as plsc

N = 50000
E = 800000
B = 1024
H = 128
L = 3
EPS = 1e-5

NB = 4
S = 12544
P = NB * S
ACC = S + 16
C = 128
NTILES = 16
EC = E // NTILES
ESB = 2000
NEB = EC // ESB
LCAP = 2176
TRASH = LCAP - 16
RPT = S // NTILES


def _agg_body(g_hbm, src_hbm, dst_hbm, out_hbm,
         acc, packed_l, sbuf, dbuf, src2d, off2d, rows, sem):
    core = lax.axis_index("c")
    sid = lax.axis_index("s")
    iota = lax.iota(jnp.int32, 16)
    padv = iota | ((S + iota) << 16)

    def fire(ci):
        cb = ci * C
        for k in range(C // 16):
            v = packed_l[pl.ds(cb + k * 16, 16)]
            src2d[0, pl.ds(k * 16, 16)] = v & 0xFFFF
            off2d[0, pl.ds(k * 16, 16)] = lax.shift_right_logical(v, 16)
        pltpu.async_copy(g_hbm.at[src2d.at[0]], rows, sem).wait()
        pltpu.sync_copy(rows, acc.at[off2d.at[0]], add=True)

    for bl in range(2):
        b = core * 2 + bl
        lo = b * S
        plsc.subcore_barrier()
        pltpu.sync_copy(g_hbm.at[pl.ds(lo + sid * RPT, RPT)],
                        acc.at[pl.ds(sid * RPT, RPT)])
        plsc.subcore_barrier()

        def block_body(eb, cur):
            base_e = sid * EC + eb * ESB
            pltpu.sync_copy(src_hbm.at[pl.ds(base_e, ESB)], sbuf)
            pltpu.sync_copy(dst_hbm.at[pl.ds(base_e, ESB)], dbuf)

            def scan_body(v, cur2):
                s16 = sbuf[pl.ds(v * 16, 16)]
                d16 = dbuf[pl.ds(v * 16, 16)]
                lov = jnp.full((16,), lo, jnp.int32)
                m = (d16 >= lov) & (d16 < lov + S)
                mi = m.astype(jnp.int32)
                packed = s16 | ((d16 - lov) << 16)
                pos = jnp.full((16,), cur2 - 1, jnp.int32) + plsc.cumsum(mi)
                pos = jnp.where(m, pos, TRASH + iota)
                plsc.store_scatter(packed_l, [pos], packed)
                return cur2 + jnp.sum(mi)

            cur = pl.loop(0, ESB // 16, init_carry=cur)(scan_body)
            nfull = cur // C
            pl.loop(0, nfull)(fire)
            rb = nfull * C
            for k in range(C // 16):
                tmp = packed_l[pl.ds(rb + k * 16, 16)]
                packed_l[pl.ds(k * 16, 16)] = tmp
            return cur - rb

        cursor = pl.loop(0, NEB, init_carry=jnp.int32(0))(block_body)

        for k in range(C // 16):
            idx = jnp.full((16,), k * 16, jnp.int32) + iota
            curv = jnp.full((16,), cursor, jnp.int32)
            idxw = jnp.where(idx >= curv, idx, TRASH + iota)
            plsc.store_scatter(packed_l, [idxw], padv)
        nlast = (cursor + (C - 1)) // C
        pl.loop(0, nlast)(fire)

        plsc.subcore_barrier()
        pltpu.sync_copy(acc.at[pl.ds(sid * RPT, RPT)],
                        out_hbm.at[pl.ds(lo + sid * RPT, RPT)])


def _sc_aggregate(g_pad, src, dst):
    mesh = plsc.VectorSubcoreMesh(core_axis_name="c", subcore_axis_name="s")
    return pl.kernel(
        _agg_body,
        out_type=jax.ShapeDtypeStruct((P, H), jnp.float32),
        mesh=mesh,
        scratch_types=[
            pltpu.VMEM_SHARED((ACC, H), jnp.float32),
            pltpu.VMEM((LCAP,), jnp.int32),
            pltpu.VMEM((ESB,), jnp.int32),
            pltpu.VMEM((ESB,), jnp.int32),
            pltpu.VMEM((1, C), jnp.int32),
            pltpu.VMEM((1, C), jnp.int32),
            pltpu.VMEM((C, H), jnp.float32),
            pltpu.SemaphoreType.DMA,
        ],
        name="sc_gcn_aggregate",
        compiler_params=pltpu.CompilerParams(needs_layout_passes=False),
    )(g_pad, src, dst)


def _node_embed_body(x_ref, w_ref, b_ref, o_ref):
    o_ref[...] = jax.nn.relu(
        jnp.dot(x_ref[...], w_ref[...], preferred_element_type=jnp.float32)
        + b_ref[...]
    )


def _node_embed(x, W, b):
    return pl.pallas_call(
        _node_embed_body,
        out_shape=jax.ShapeDtypeStruct((N, H), jnp.float32),
    )(x, W, b[None, :])


def _bn(x, g, b):
    mu = jnp.mean(x, axis=0)
    var = jnp.var(x, axis=0)
    return (x - mu) / jnp.sqrt(var + EPS) * g + b


def kernel(x, edge_index, batch, gene_features, params):
    src, dst = edge_index[0], edge_index[1]
    h = _node_embed(x, params['node_W'], params['node_b'])
    deg = jnp.zeros((N,), jnp.float32).at[dst].add(1.0) + 1.0
    dinv = 1.0 / jnp.sqrt(deg)
    for i in range(L):
        g = (h @ params['gcn_W'][i]) * dinv[:, None]
        g_pad = jnp.pad(g, ((0, P - N), (0, 0)))
        agg = _sc_aggregate(g_pad, src, dst)[:N]
        hn = agg * dinv[:, None] + params['gcn_b'][i]
        hn = _bn(hn, params['bn_g'][i], params['bn_b'][i])
        hn = jax.nn.relu(hn)
        h = h + hn if i > 0 else hn
    ones = jnp.ones((N,), jnp.float32)
    counts = jax.ops.segment_sum(ones, batch, num_segments=B)
    sums = jax.ops.segment_sum(h, batch, num_segments=B)
    mean_pool = sums / jnp.clip(counts, 1.0)[:, None]
    max_pool = jax.ops.segment_max(h, batch, num_segments=B)
    max_pool = jnp.where(jnp.isfinite(max_pool), max_pool, 0.0)
    graph_repr = jnp.concatenate([mean_pool, max_pool], axis=1)
    d1 = jax.nn.relu(graph_repr @ params['proj_W1'] + params['proj_b1'])
    drug_embedding = d1 @ params['proj_W2'] + params['proj_b2']
    g1 = gene_features @ params['gene_W1'] + params['gene_b1']
    g1 = jax.nn.relu(_bn(g1, params['gene_bn_g'], params['gene_bn_b']))
    gene_embedding = jax.nn.relu(g1 @ params['gene_W2'] + params['gene_b2'])
    combined = jnp.concatenate([drug_embedding, gene_embedding], axis=1)
    p1 = combined @ params['head_W1'] + params['head_b1']
    p1 = jax.nn.relu(_bn(p1, params['head_bn_g'], params['head_bn_b']))
    p2 = jax.nn.relu(p1 @ params['head_W2'] + params['head_b2'])
    predictions = p2 @ params['head_W3'] + params['head_b3']
    return predictions


# speedup vs baseline: 9.1233x; 1.0168x over previous
import jax
import jax.numpy as jnp
from jax import lax
from jax.experimental import pallas as pl
from jax.experimental.pallas import tpu as pltpu
from jax.experimental.pallas import tpu_sc as plsc

N = 50000
E = 800000
B = 1024
H = 128
L = 3
EPS = 1e-5

NB = 4
S = 12544
P = NB * S
ACC = S + 16
C = 64
NTILES = 16
EC = E // NTILES
ESB = 2000
NEB = EC // ESB
LCAP = 2112
TRASH = LCAP - 16
RPT = S // NTILES


def _agg_body(g_hbm, src_hbm, dst_hbm, out_hbm,
         acc, packed_l, sbuf, dbuf,
         s2a, o2a, s2b, o2b, rows_a, rows_b,
         semga, semgb, semsa, semsb):
    core = lax.axis_index("c")
    sid = lax.axis_index("s")
    iota = lax.iota(jnp.int32, 16)
    padv = iota | ((S + iota) << 16)

    def _prep(ci, s2d, o2d):
        cb = ci * C
        for k in range(C // 16):
            v = packed_l[pl.ds(cb + k * 16, 16)]
            s2d[0, pl.ds(k * 16, 16)] = v & 0xFFFF
            o2d[0, pl.ds(k * 16, 16)] = lax.shift_right_logical(v, 16)

    def fire(ci):
        _prep(ci, s2a, o2a)
        pltpu.async_copy(g_hbm.at[s2a.at[0]], rows_a, semga).wait()
        pltpu.sync_copy(rows_a, acc.at[o2a.at[0]], add=True)

    def fire_pair(p):
        _prep(2 * p, s2a, o2a)
        ga = pltpu.async_copy(g_hbm.at[s2a.at[0]], rows_a, semga)
        _prep(2 * p + 1, s2b, o2b)
        gb = pltpu.async_copy(g_hbm.at[s2b.at[0]], rows_b, semgb)
        ga.wait()
        sa = pltpu.async_copy(rows_a, acc.at[o2a.at[0]], semsa, add=True)
        gb.wait()
        sb = pltpu.async_copy(rows_b, acc.at[o2b.at[0]], semsb, add=True)
        sa.wait()
        sb.wait()

    for bl in range(2):
        b = core * 2 + bl
        lo = b * S
        plsc.subcore_barrier()
        pltpu.sync_copy(g_hbm.at[pl.ds(lo + sid * RPT, RPT)],
                        acc.at[pl.ds(sid * RPT, RPT)])
        plsc.subcore_barrier()

        def block_body(eb, cur):
            base_e = sid * EC + eb * ESB
            pltpu.sync_copy(src_hbm.at[pl.ds(base_e, ESB)], sbuf)
            pltpu.sync_copy(dst_hbm.at[pl.ds(base_e, ESB)], dbuf)

            def scan_body(v, cur2):
                s16 = sbuf[pl.ds(v * 16, 16)]
                d16 = dbuf[pl.ds(v * 16, 16)]
                lov = jnp.full((16,), lo, jnp.int32)
                m = (d16 >= lov) & (d16 < lov + S)
                mi = m.astype(jnp.int32)
                packed = s16 | ((d16 - lov) << 16)
                pos = jnp.full((16,), cur2 - 1, jnp.int32) + plsc.cumsum(mi)
                pos = jnp.where(m, pos, TRASH + iota)
                plsc.store_scatter(packed_l, [pos], packed)
                return cur2 + jnp.sum(mi)

            cur = pl.loop(0, ESB // 16, init_carry=cur)(scan_body)
            nfull = cur // C
            pl.loop(0, nfull // 2)(fire_pair)

            @pl.when(nfull % 2 == 1)
            def _():
                fire(nfull - 1)

            rb = nfull * C
            for k in range(C // 16):
                tmp = packed_l[pl.ds(rb + k * 16, 16)]
                packed_l[pl.ds(k * 16, 16)] = tmp
            return cur - rb

        cursor = pl.loop(0, NEB, init_carry=jnp.int32(0))(block_body)

        for k in range(C // 16):
            idx = jnp.full((16,), k * 16, jnp.int32) + iota
            curv = jnp.full((16,), cursor, jnp.int32)
            idxw = jnp.where(idx >= curv, idx, TRASH + iota)
            plsc.store_scatter(packed_l, [idxw], padv)
        nlast = (cursor + (C - 1)) // C

        @pl.when(nlast == 1)
        def _():
            fire(0)

        plsc.subcore_barrier()
        pltpu.sync_copy(acc.at[pl.ds(sid * RPT, RPT)],
                        out_hbm.at[pl.ds(lo + sid * RPT, RPT)])


def _sc_aggregate(g_pad, src, dst):
    mesh = plsc.VectorSubcoreMesh(core_axis_name="c", subcore_axis_name="s")
    return pl.kernel(
        _agg_body,
        out_type=jax.ShapeDtypeStruct((P, H), jnp.float32),
        mesh=mesh,
        scratch_types=[
            pltpu.VMEM_SHARED((ACC, H), jnp.float32),
            pltpu.VMEM((LCAP,), jnp.int32),
            pltpu.VMEM((ESB,), jnp.int32),
            pltpu.VMEM((ESB,), jnp.int32),
            pltpu.VMEM((1, C), jnp.int32),
            pltpu.VMEM((1, C), jnp.int32),
            pltpu.VMEM((1, C), jnp.int32),
            pltpu.VMEM((1, C), jnp.int32),
            pltpu.VMEM((C, H), jnp.float32),
            pltpu.VMEM((C, H), jnp.float32),
            pltpu.SemaphoreType.DMA,
            pltpu.SemaphoreType.DMA,
            pltpu.SemaphoreType.DMA,
            pltpu.SemaphoreType.DMA,
        ],
        name="sc_gcn_aggregate",
        compiler_params=pltpu.CompilerParams(needs_layout_passes=False),
    )(g_pad, src, dst)


def _node_embed_body(x_ref, w_ref, b_ref, o_ref):
    o_ref[...] = jax.nn.relu(
        jnp.dot(x_ref[...], w_ref[...], preferred_element_type=jnp.float32)
        + b_ref[...]
    )


def _node_embed(x, W, b):
    return pl.pallas_call(
        _node_embed_body,
        out_shape=jax.ShapeDtypeStruct((N, H), jnp.float32),
    )(x, W, b[None, :])


def _bn(x, g, b):
    mu = jnp.mean(x, axis=0)
    var = jnp.var(x, axis=0)
    return (x - mu) / jnp.sqrt(var + EPS) * g + b


def kernel(x, edge_index, batch, gene_features, params):
    src, dst = edge_index[0], edge_index[1]
    h = _node_embed(x, params['node_W'], params['node_b'])
    deg = jnp.zeros((N,), jnp.float32).at[dst].add(1.0) + 1.0
    dinv = 1.0 / jnp.sqrt(deg)
    for i in range(L):
        g = (h @ params['gcn_W'][i]) * dinv[:, None]
        g_pad = jnp.pad(g, ((0, P - N), (0, 0)))
        agg = _sc_aggregate(g_pad, src, dst)[:N]
        hn = agg * dinv[:, None] + params['gcn_b'][i]
        hn = _bn(hn, params['bn_g'][i], params['bn_b'][i])
        hn = jax.nn.relu(hn)
        h = h + hn if i > 0 else hn
    ones = jnp.ones((N,), jnp.float32)
    counts = jax.ops.segment_sum(ones, batch, num_segments=B)
    sums = jax.ops.segment_sum(h, batch, num_segments=B)
    mean_pool = sums / jnp.clip(counts, 1.0)[:, None]
    max_pool = jax.ops.segment_max(h, batch, num_segments=B)
    max_pool = jnp.where(jnp.isfinite(max_pool), max_pool, 0.0)
    graph_repr = jnp.concatenate([mean_pool, max_pool], axis=1)
    d1 = jax.nn.relu(graph_repr @ params['proj_W1'] + params['proj_b1'])
    drug_embedding = d1 @ params['proj_W2'] + params['proj_b2']
    g1 = gene_features @ params['gene_W1'] + params['gene_b1']
    g1 = jax.nn.relu(_bn(g1, params['gene_bn_g'], params['gene_bn_b']))
    gene_embedding = jax.nn.relu(g1 @ params['gene_W2'] + params['gene_b2'])
    combined = jnp.concatenate([drug_embedding, gene_embedding], axis=1)
    p1 = combined @ params['head_W1'] + params['head_b1']
    p1 = jax.nn.relu(_bn(p1, params['head_bn_g'], params['head_bn_b']))
    p2 = jax.nn.relu(p1 @ params['head_W2'] + params['head_b2'])
    predictions = p2 @ params['head_W3'] + params['head_b3']
    return predictions


# speedup vs baseline: 9.2381x; 1.0126x over previous
import jax
import jax.numpy as jnp
from jax import lax
from jax.experimental import pallas as pl
from jax.experimental.pallas import tpu as pltpu
from jax.experimental.pallas import tpu_sc as plsc

N = 50000
E = 800000
B = 1024
H = 128
L = 3
EPS = 1e-5

NB = 4
S = 12544
P = NB * S
ACC = S + 16
C = 64
NTILES = 16
EC = E // NTILES
ESB = 2000
NEB = EC // ESB
LCAP = 2112
LCAP2 = 50112
TRASH = LCAP - 16
RPT = S // NTILES


def _bin_body(src_hbm, dst_hbm, lists, cnts, packed_l, sbuf, dbuf, cntbuf):
    core = lax.axis_index("c")
    sid = lax.axis_index("s")
    iota = lax.iota(jnp.int32, 16)
    padv = iota | ((S + iota) << 16)

    for bl in range(2):
        b = core * 2 + bl
        lo = b * S
        lbase = (b * NTILES + sid) * LCAP2

        def block_body(eb, carry):
            cur, hcur = carry
            hcur = pl.multiple_of(hcur, C)
            base_e = sid * EC + eb * ESB
            pltpu.sync_copy(src_hbm.at[pl.ds(base_e, ESB)], sbuf)
            pltpu.sync_copy(dst_hbm.at[pl.ds(base_e, ESB)], dbuf)

            def scan_body(v, cur2):
                s16 = sbuf[pl.ds(v * 16, 16)]
                d16 = dbuf[pl.ds(v * 16, 16)]
                lov = jnp.full((16,), lo, jnp.int32)
                m = (d16 >= lov) & (d16 < lov + S)
                mi = m.astype(jnp.int32)
                packed = s16 | ((d16 - lov) << 16)
                pos = jnp.full((16,), cur2 - 1, jnp.int32) + plsc.cumsum(mi)
                pos = jnp.where(m, pos, TRASH + iota)
                plsc.store_scatter(packed_l, [pos], packed)
                return cur2 + jnp.sum(mi)

            cur = pl.loop(0, ESB // 16, init_carry=cur)(scan_body)
            nfull = cur // C

            def flush(ci):
                pltpu.sync_copy(
                    packed_l.at[pl.ds(ci * C, C)],
                    lists.at[pl.ds(lbase + hcur + ci * C, C)])

            pl.loop(0, nfull)(flush)
            rb = nfull * C
            for k in range(C // 16):
                tmp = packed_l[pl.ds(rb + k * 16, 16)]
                packed_l[pl.ds(k * 16, 16)] = tmp
            return cur - rb, hcur + rb

        cursor, hcur = pl.loop(
            0, NEB, init_carry=(jnp.int32(0), jnp.int32(0)))(block_body)

        for k in range(C // 16):
            idx = jnp.full((16,), k * 16, jnp.int32) + iota
            curv = jnp.full((16,), cursor, jnp.int32)
            idxw = jnp.where(idx >= curv, idx, TRASH + iota)
            plsc.store_scatter(packed_l, [idxw], padv)
        nlast = (cursor + (C - 1)) // C

        hcur = pl.multiple_of(hcur, C)

        @pl.when(nlast == 1)
        def _():
            pltpu.sync_copy(packed_l.at[pl.ds(0, C)],
                            lists.at[pl.ds(lbase + hcur, C)])

        cntbuf[...] = jnp.full((16,), hcur + nlast * C, jnp.int32)
        pltpu.sync_copy(cntbuf, cnts.at[pl.ds((b * NTILES + sid) * 16, 16)])


def _sc_bin(src, dst):
    mesh = plsc.VectorSubcoreMesh(core_axis_name="c", subcore_axis_name="s")
    return pl.kernel(
        _bin_body,
        out_type=(
            jax.ShapeDtypeStruct((NB * NTILES * LCAP2,), jnp.int32),
            jax.ShapeDtypeStruct((NB * NTILES * 16,), jnp.int32),
        ),
        mesh=mesh,
        scratch_types=[
            pltpu.VMEM((LCAP,), jnp.int32),
            pltpu.VMEM((ESB,), jnp.int32),
            pltpu.VMEM((ESB,), jnp.int32),
            pltpu.VMEM((16,), jnp.int32),
        ],
        name="sc_gcn_bin",
        compiler_params=pltpu.CompilerParams(needs_layout_passes=False),
    )(src, dst)


def _agg_body(g_hbm, lists, cnts, out_hbm,
              acc, packed_l, cntv,
              s2a, o2a, s2b, o2b, rows_a, rows_b,
              semga, semgb, semsa, semsb):
    core = lax.axis_index("c")
    sid = lax.axis_index("s")

    def _prep(off, s2d, o2d):
        for k in range(C // 16):
            v = packed_l[pl.ds(off + k * 16, 16)]
            s2d[0, pl.ds(k * 16, 16)] = v & 0xFFFF
            o2d[0, pl.ds(k * 16, 16)] = lax.shift_right_logical(v, 16)

    for bl in range(2):
        b = core * 2 + bl
        lo = b * S
        lbase = (b * NTILES + sid) * LCAP2
        plsc.subcore_barrier()
        pltpu.sync_copy(g_hbm.at[pl.ds(lo + sid * RPT, RPT)],
                        acc.at[pl.ds(sid * RPT, RPT)])
        plsc.subcore_barrier()

        pltpu.sync_copy(cnts.at[pl.ds((b * NTILES + sid) * 16, 16)], cntv)
        total = jnp.max(cntv[...])
        nfull = pl.multiple_of(total, C) // C

        def fire_pair(p):
            pltpu.sync_copy(lists.at[pl.ds(lbase + 2 * p * C, 2 * C)],
                            packed_l)
            _prep(0, s2a, o2a)
            ga = pltpu.async_copy(g_hbm.at[s2a.at[0]], rows_a, semga)
            _prep(C, s2b, o2b)
            gb = pltpu.async_copy(g_hbm.at[s2b.at[0]], rows_b, semgb)
            ga.wait()
            sa = pltpu.async_copy(rows_a, acc.at[o2a.at[0]], semsa, add=True)
            gb.wait()
            sb = pltpu.async_copy(rows_b, acc.at[o2b.at[0]], semsb, add=True)
            sa.wait()
            sb.wait()

        pl.loop(0, nfull // 2)(fire_pair)

        @pl.when(nfull % 2 == 1)
        def _():
            pltpu.sync_copy(lists.at[pl.ds(lbase + (nfull - 1) * C, C)],
                            packed_l.at[pl.ds(0, C)])
            _prep(0, s2a, o2a)
            pltpu.async_copy(g_hbm.at[s2a.at[0]], rows_a, semga).wait()
            pltpu.sync_copy(rows_a, acc.at[o2a.at[0]], add=True)

        plsc.subcore_barrier()
        pltpu.sync_copy(acc.at[pl.ds(sid * RPT, RPT)],
                        out_hbm.at[pl.ds(lo + sid * RPT, RPT)])


def _sc_aggregate(g_pad, lists, cnts):
    mesh = plsc.VectorSubcoreMesh(core_axis_name="c", subcore_axis_name="s")
    return pl.kernel(
        _agg_body,
        out_type=jax.ShapeDtypeStruct((P, H), jnp.float32),
        mesh=mesh,
        scratch_types=[
            pltpu.VMEM_SHARED((ACC, H), jnp.float32),
            pltpu.VMEM((2 * C,), jnp.int32),
            pltpu.VMEM((16,), jnp.int32),
            pltpu.VMEM((1, C), jnp.int32),
            pltpu.VMEM((1, C), jnp.int32),
            pltpu.VMEM((1, C), jnp.int32),
            pltpu.VMEM((1, C), jnp.int32),
            pltpu.VMEM((C, H), jnp.float32),
            pltpu.VMEM((C, H), jnp.float32),
            pltpu.SemaphoreType.DMA,
            pltpu.SemaphoreType.DMA,
            pltpu.SemaphoreType.DMA,
            pltpu.SemaphoreType.DMA,
        ],
        name="sc_gcn_aggregate",
        compiler_params=pltpu.CompilerParams(needs_layout_passes=False),
    )(g_pad, lists, cnts)


def _node_embed_body(x_ref, w_ref, b_ref, o_ref):
    o_ref[...] = jax.nn.relu(
        jnp.dot(x_ref[...], w_ref[...], preferred_element_type=jnp.float32)
        + b_ref[...]
    )


def _node_embed(x, W, b):
    return pl.pallas_call(
        _node_embed_body,
        out_shape=jax.ShapeDtypeStruct((N, H), jnp.float32),
    )(x, W, b[None, :])


def _bn(x, g, b):
    mu = jnp.mean(x, axis=0)
    var = jnp.var(x, axis=0)
    return (x - mu) / jnp.sqrt(var + EPS) * g + b


def kernel(x, edge_index, batch, gene_features, params):
    src, dst = edge_index[0], edge_index[1]
    lists, cnts = _sc_bin(src, dst)
    h = _node_embed(x, params['node_W'], params['node_b'])
    deg = jnp.zeros((N,), jnp.float32).at[dst].add(1.0) + 1.0
    dinv = 1.0 / jnp.sqrt(deg)
    for i in range(L):
        g = (h @ params['gcn_W'][i]) * dinv[:, None]
        g_pad = jnp.pad(g, ((0, P - N), (0, 0)))
        agg = _sc_aggregate(g_pad, lists, cnts)[:N]
        hn = agg * dinv[:, None] + params['gcn_b'][i]
        hn = _bn(hn, params['bn_g'][i], params['bn_b'][i])
        hn = jax.nn.relu(hn)
        h = h + hn if i > 0 else hn
    ones = jnp.ones((N,), jnp.float32)
    counts = jax.ops.segment_sum(ones, batch, num_segments=B)
    sums = jax.ops.segment_sum(h, batch, num_segments=B)
    mean_pool = sums / jnp.clip(counts, 1.0)[:, None]
    max_pool = jax.ops.segment_max(h, batch, num_segments=B)
    max_pool = jnp.where(jnp.isfinite(max_pool), max_pool, 0.0)
    graph_repr = jnp.concatenate([mean_pool, max_pool], axis=1)
    d1 = jax.nn.relu(graph_repr @ params['proj_W1'] + params['proj_b1'])
    drug_embedding = d1 @ params['proj_W2'] + params['proj_b2']
    g1 = gene_features @ params['gene_W1'] + params['gene_b1']
    g1 = jax.nn.relu(_bn(g1, params['gene_bn_g'], params['gene_bn_b']))
    gene_embedding = jax.nn.relu(g1 @ params['gene_W2'] + params['gene_b2'])
    combined = jnp.concatenate([drug_embedding, gene_embedding], axis=1)
    p1 = combined @ params['head_W1'] + params['head_b1']
    p1 = jax.nn.relu(_bn(p1, params['head_bn_g'], params['head_bn_b']))
    p2 = jax.nn.relu(p1 @ params['head_W2'] + params['head_b2'])
    predictions = p2 @ params['head_W3'] + params['head_b3']
    return predictions
